# Initial kernel scaffold; baseline (speedup 1.0000x reference)
import functools
import math

import jax
import jax.numpy as jnp
from jax import lax
from jax.experimental import pallas as pl
from jax.experimental.pallas import tpu as pltpu
from jax.experimental.pallas import tpu_sc as plsc

B, C, H, W = 2, 128, 256, 256
HW = H * W
N, C_CAM = 20000, 128
HEADS, DHEAD = 4, 32
HD = HEADS * DHEAD
GAMMA = 0.08
SCALE = 1.0 / math.sqrt(DHEAD)

NTILES = 32
ROWS = B * C
ROWS_PER_TILE = ROWS // NTILES
NC = 2048
NSTEPS = (N + NC - 1) // NC
PC = 2048

_SC_MESH = plsc.VectorSubcoreMesh(core_axis_name="c", subcore_axis_name="s")


def _lin_body(ii_ref, jj_ref, lin_ref):
    i = jnp.clip(ii_ref[...], 0, H - 1)
    j = jnp.clip(jj_ref[...], 0, W - 1)
    lin_ref[...] = i * W + j


def _lin_call(ii3, jj3):
    return pl.pallas_call(
        _lin_body,
        grid=(B,),
        in_specs=[
            pl.BlockSpec((1, 1, N), lambda b: (b, 0, 0)),
            pl.BlockSpec((1, 1, N), lambda b: (b, 0, 0)),
        ],
        out_specs=pl.BlockSpec((1, 1, N), lambda b: (b, 0, 0)),
        out_shape=jax.ShapeDtypeStruct((B, 1, N), jnp.int32),
    )(ii3, jj3)


def _sc_gather_body(lidar_hbm, lin_hbm, g_hbm, row_v, idx_v, out_v):
    cid = lax.axis_index("c")
    sid = lax.axis_index("s")
    wid = sid * 2 + cid
    b = wid // (NTILES // B)
    pltpu.sync_copy(lin_hbm.at[pl.ds(b * N, N)], idx_v)

    def body(i, _):
        off = i * 16
        ix = idx_v[pl.ds(off, 16)]
        out_v[pl.ds(off, 16)] = plsc.load_gather(row_v, [ix])
        return 0

    for k in range(ROWS_PER_TILE):
        r = wid * ROWS_PER_TILE + k
        pltpu.sync_copy(lidar_hbm.at[pl.ds(r * HW, HW)], row_v)
        lax.fori_loop(0, N // 16, body, 0, unroll=8)
        pltpu.sync_copy(out_v, g_hbm.at[pl.ds(r * N, N)])


def _run_sc_gather(lidar_flat, lin_flat):
    fn = pl.kernel(
        _sc_gather_body,
        out_type=jax.ShapeDtypeStruct((ROWS * N,), jnp.float32),
        mesh=_SC_MESH,
        scratch_types=[
            pltpu.VMEM((HW,), jnp.float32),
            pltpu.VMEM((N,), jnp.int32),
            pltpu.VMEM((N,), jnp.float32),
        ],
    )
    return fn(lidar_flat, lin_flat)


def _sc_scatter_body(ot_hbm, lin_hbm, delta_hbm, hits_hbm, acc_v, idx_v, dat_v):
    cid = lax.axis_index("c")
    sid = lax.axis_index("s")
    wid = sid * 2 + cid
    b = wid // (NTILES // B)
    pltpu.sync_copy(lin_hbm.at[pl.ds(b * N, N)], idx_v)
    zero16 = jnp.zeros((16,), jnp.float32)
    ones16 = jnp.ones((16,), jnp.float32)

    def zero_body(i, _):
        acc_v[pl.ds(i * 16, 16)] = zero16
        return 0

    def add_body(i, _):
        off = i * 16
        ix = idx_v[pl.ds(off, 16)]
        d = dat_v[pl.ds(off, 16)]
        plsc.addupdate_scatter(acc_v, [ix], d)
        return 0

    def hit_body(i, _):
        ix = idx_v[pl.ds(i * 16, 16)]
        plsc.addupdate_scatter(acc_v, [ix], ones16)
        return 0

    for k in range(ROWS_PER_TILE):
        r = wid * ROWS_PER_TILE + k
        lax.fori_loop(0, HW // 16, zero_body, 0, unroll=8)
        pltpu.sync_copy(ot_hbm.at[pl.ds(r * N, N)], dat_v)
        lax.fori_loop(0, N // 16, add_body, 0, unroll=8)
        pltpu.sync_copy(acc_v, delta_hbm.at[pl.ds(r * HW, HW)])

    @pl.when(jnp.logical_or(wid == 0, wid == NTILES // B))
    def _():
        lax.fori_loop(0, HW // 16, zero_body, 0, unroll=8)
        lax.fori_loop(0, N // 16, hit_body, 0, unroll=8)
        pltpu.sync_copy(acc_v, hits_hbm.at[pl.ds(b * HW, HW)])


def _run_sc_scatter(ot_flat, lin_flat):
    fn = pl.kernel(
        _sc_scatter_body,
        out_type=(
            jax.ShapeDtypeStruct((ROWS * HW,), jnp.float32),
            jax.ShapeDtypeStruct((B * HW,), jnp.float32),
        ),
        mesh=_SC_MESH,
        scratch_types=[
            pltpu.VMEM((HW,), jnp.float32),
            pltpu.VMEM((N,), jnp.int32),
            pltpu.VMEM((N,), jnp.float32),
        ],
    )
    return fn(ot_flat, lin_flat)


def _head_onehot():
    col = lax.broadcasted_iota(jnp.int32, (HEADS, HD), 1) // DHEAD
    row = lax.broadcasted_iota(jnp.int32, (HEADS, HD), 0)
    return (col == row).astype(jnp.float32)


def _tc1_body(g_ref, tok_ref, wq_ref, bq_ref, wk_ref, s_ref):
    g = g_ref[0]
    tok = tok_ref[0]
    q = jnp.dot(wq_ref[...], g, preferred_element_type=jnp.float32) + bq_ref[...]
    k = lax.dot_general(wk_ref[...], tok, (((1,), (1,)), ((), ())),
                        preferred_element_type=jnp.float32)
    s = jnp.dot(_head_onehot(), q * k, preferred_element_type=jnp.float32)
    s_ref[0] = s * SCALE


def _tc1_call(g3, tok, Wq, bq2, Wk):
    return pl.pallas_call(
        _tc1_body,
        grid=(B, NSTEPS),
        in_specs=[
            pl.BlockSpec((1, C, NC), lambda b, n: (b, 0, n)),
            pl.BlockSpec((1, NC, C_CAM), lambda b, n: (b, n, 0)),
            pl.BlockSpec((HD, C), lambda b, n: (0, 0)),
            pl.BlockSpec((HD, 1), lambda b, n: (0, 0)),
            pl.BlockSpec((HD, C_CAM), lambda b, n: (0, 0)),
        ],
        out_specs=pl.BlockSpec((1, HEADS, NC), lambda b, n: (b, 0, n)),
        out_shape=jax.ShapeDtypeStruct((B, HEADS, N), jnp.float32),
    )(g3, tok, Wq, bq2, Wk)


def _softmax_body(s_ref, gw_ref, p_ref):
    s = s_ref[0]
    m = jnp.max(s, axis=-1, keepdims=True)
    e = jnp.exp(s - m)
    z = jnp.sum(e, axis=-1, keepdims=True)
    p_ref[0] = e / z * gw_ref[0]


def _softmax_call(s3, gw3):
    return pl.pallas_call(
        _softmax_body,
        grid=(B,),
        in_specs=[
            pl.BlockSpec((1, HEADS, N), lambda b: (b, 0, 0)),
            pl.BlockSpec((1, 1, N), lambda b: (b, 0, 0)),
        ],
        out_specs=pl.BlockSpec((1, HEADS, N), lambda b: (b, 0, 0)),
        out_shape=jax.ShapeDtypeStruct((B, HEADS, N), jnp.float32),
    )(s3, gw3)


def _tc2_body(tok_ref, p_ref, wv_ref, wo_ref, ot_ref, cs_ref):
    nstep = pl.program_id(1)
    tok = tok_ref[0]
    v = lax.dot_general(wv_ref[...], tok, (((1,), (1,)), ((), ())),
                        preferred_element_type=jnp.float32)
    p = p_ref[0]
    pe = lax.dot_general(_head_onehot(), p, (((0,), (0,)), ((), ())),
                         preferred_element_type=jnp.float32)
    lane = lax.broadcasted_iota(jnp.int32, (HD, NC), 1) + nstep * NC
    fused = jnp.where(lane < N, pe * v, 0.0)
    ot_ref[0] = jnp.dot(wo_ref[...], fused, preferred_element_type=jnp.float32)
    cs = jnp.dot(wo_ref[...], jnp.sum(fused, axis=1, keepdims=True),
                 preferred_element_type=jnp.float32)

    @pl.when(nstep == 0)
    def _():
        cs_ref[0] = cs

    @pl.when(nstep > 0)
    def _():
        cs_ref[0] += cs


def _tc2_call(tok, p3, Wv, Wo):
    return pl.pallas_call(
        _tc2_body,
        grid=(B, NSTEPS),
        in_specs=[
            pl.BlockSpec((1, NC, C_CAM), lambda b, n: (b, n, 0)),
            pl.BlockSpec((1, HEADS, NC), lambda b, n: (b, 0, n)),
            pl.BlockSpec((HD, C_CAM), lambda b, n: (0, 0)),
            pl.BlockSpec((C, HD), lambda b, n: (0, 0)),
        ],
        out_specs=[
            pl.BlockSpec((1, C, NC), lambda b, n: (b, 0, n)),
            pl.BlockSpec((1, C, 1), lambda b, n: (b, 0, 0)),
        ],
        out_shape=[
            jax.ShapeDtypeStruct((B, C, N), jnp.float32),
            jax.ShapeDtypeStruct((B, C, 1), jnp.float32),
        ],
    )(tok, p3, Wv, Wo)


def _combine_body(lid_ref, dl_ref, al_ref, ht_ref, hf_ref, cs_ref, o_ref):
    hits_full = hf_ref[0]
    nhit = jnp.sum((hits_full > 0.0).astype(jnp.float32))
    mean = cs_ref[0] / (nhit + 1e-6)
    mask = (ht_ref[0] > 0.0).astype(jnp.float32)
    d = dl_ref[0] - mean * mask
    o_ref[0] = lid_ref[0] + d * (al_ref[0] * GAMMA)


def _combine_call(lidar3, delta3, alpha3, hits3, cs3):
    return pl.pallas_call(
        _combine_body,
        grid=(B, HW // PC),
        in_specs=[
            pl.BlockSpec((1, C, PC), lambda b, p: (b, 0, p)),
            pl.BlockSpec((1, C, PC), lambda b, p: (b, 0, p)),
            pl.BlockSpec((1, 1, PC), lambda b, p: (0, 0, p)),
            pl.BlockSpec((1, 1, PC), lambda b, p: (b, 0, p)),
            pl.BlockSpec((1, 1, HW), lambda b, p: (b, 0, 0)),
            pl.BlockSpec((1, C, 1), lambda b, p: (b, 0, 0)),
        ],
        out_specs=pl.BlockSpec((1, C, PC), lambda b, p: (b, 0, p)),
        out_shape=jax.ShapeDtypeStruct((B, C, HW), jnp.float32),
    )(lidar3, delta3, alpha3, hits3, cs3)


def kernel(lidar_bev, cam_bev_tokens, cam_bev_indices, gate_weights,
           range_alpha, Wq, bq, Wk, Wv, Wo):
    lidar3 = lidar_bev.reshape(B, C, HW)
    lidar_flat = lidar_bev.reshape(B * C * HW)
    ind = cam_bev_indices.astype(jnp.int32)
    ii3 = ind[..., 0].reshape(B, 1, N)
    jj3 = ind[..., 1].reshape(B, 1, N)
    gw3 = gate_weights.reshape(B, 1, N)
    alpha3 = range_alpha.reshape(1, 1, HW)
    bq2 = bq.reshape(HD, 1)

    lin_flat = _lin_call(ii3, jj3).reshape(B * N)
    g3 = _run_sc_gather(lidar_flat, lin_flat).reshape(B, C, N)
    s3 = _tc1_call(g3, cam_bev_tokens, Wq, bq2, Wk)
    p3 = _softmax_call(s3, gw3)
    ot3, cs3 = _tc2_call(cam_bev_tokens, p3, Wv, Wo)
    delta_flat, hits_flat = _run_sc_scatter(ot3.reshape(B * C * N), lin_flat)
    out3 = _combine_call(lidar3, delta_flat.reshape(B, C, HW), alpha3,
                         hits_flat.reshape(B, 1, HW), cs3)
    return out3.reshape(B, C, H, W)


# speedup vs baseline: 6.4215x; 6.4215x over previous
import functools
import math

import jax
import jax.numpy as jnp
from jax import lax
from jax.experimental import pallas as pl
from jax.experimental.pallas import tpu as pltpu
from jax.experimental.pallas import tpu_sc as plsc

B, C, H, W = 2, 128, 256, 256
HW = H * W
N, C_CAM = 20000, 128
HEADS, DHEAD = 4, 32
HD = HEADS * DHEAD
GAMMA = 0.08
SCALE = 1.0 / math.sqrt(DHEAD)

NTILES = 32
ROWS = B * C
ROWS_PER_TILE = ROWS // NTILES
NC = 2048
NSTEPS = (N + NC - 1) // NC
PC = 2048

_SC_MESH = plsc.VectorSubcoreMesh(core_axis_name="c", subcore_axis_name="s")
_SC_PARAMS = pltpu.CompilerParams(needs_layout_passes=False)


def _lin_body(ii_ref, jj_ref, lin_ref):
    i = jnp.clip(ii_ref[...], 0, H - 1)
    j = jnp.clip(jj_ref[...], 0, W - 1)
    lin_ref[...] = i * W + j


def _lin_call(ii3, jj3):
    return pl.pallas_call(
        _lin_body,
        grid=(B,),
        in_specs=[
            pl.BlockSpec((1, 1, N), lambda b: (b, 0, 0)),
            pl.BlockSpec((1, 1, N), lambda b: (b, 0, 0)),
        ],
        out_specs=pl.BlockSpec((1, 1, N), lambda b: (b, 0, 0)),
        out_shape=jax.ShapeDtypeStruct((B, 1, N), jnp.int32),
    )(ii3, jj3)


def _sc_gather_body(lidar_hbm, lin_hbm, g_hbm, row_v, idx_v, out_v):
    cid = lax.axis_index("c")
    sid = lax.axis_index("s")
    wid = sid * 2 + cid
    b = wid // (NTILES // B)
    pltpu.sync_copy(lin_hbm.at[pl.ds(b * N, N)], idx_v)

    def body(i, _):
        off = i * 16
        ix = idx_v[pl.ds(off, 16)]
        out_v[pl.ds(off, 16)] = plsc.load_gather(row_v, [ix])
        return 0

    for k in range(ROWS_PER_TILE):
        r = wid * ROWS_PER_TILE + k
        pltpu.sync_copy(lidar_hbm.at[pl.ds(r * HW, HW)], row_v)
        lax.fori_loop(0, N // 16, body, 0, unroll=8)
        pltpu.sync_copy(out_v, g_hbm.at[pl.ds(r * N, N)])


def _run_sc_gather(lidar_flat, lin_flat):
    fn = pl.kernel(
        _sc_gather_body,
        out_type=jax.ShapeDtypeStruct((ROWS * N,), jnp.float32),
        mesh=_SC_MESH,
        compiler_params=_SC_PARAMS,
        scratch_types=[
            pltpu.VMEM((HW,), jnp.float32),
            pltpu.VMEM((N,), jnp.int32),
            pltpu.VMEM((N,), jnp.float32),
        ],
    )
    return fn(lidar_flat, lin_flat)


def _sc_scatter_body(ot_hbm, lin_hbm, delta_hbm, hits_hbm, acc_v, idx_v, dat_v):
    cid = lax.axis_index("c")
    sid = lax.axis_index("s")
    wid = sid * 2 + cid
    b = wid // (NTILES // B)
    pltpu.sync_copy(lin_hbm.at[pl.ds(b * N, N)], idx_v)
    zero16 = jnp.zeros((16,), jnp.float32)
    ones16 = jnp.ones((16,), jnp.float32)

    def zero_body(i, _):
        acc_v[pl.ds(i * 16, 16)] = zero16
        return 0

    def add_body(i, _):
        off = i * 16
        ix = idx_v[pl.ds(off, 16)]
        d = dat_v[pl.ds(off, 16)]
        plsc.addupdate_scatter(acc_v, [ix], d)
        return 0

    def hit_body(i, _):
        ix = idx_v[pl.ds(i * 16, 16)]
        plsc.addupdate_scatter(acc_v, [ix], ones16)
        return 0

    for k in range(ROWS_PER_TILE):
        r = wid * ROWS_PER_TILE + k
        lax.fori_loop(0, HW // 16, zero_body, 0, unroll=8)
        pltpu.sync_copy(ot_hbm.at[pl.ds(r * N, N)], dat_v)
        lax.fori_loop(0, N // 16, add_body, 0, unroll=8)
        pltpu.sync_copy(acc_v, delta_hbm.at[pl.ds(r * HW, HW)])

    @pl.when(jnp.logical_or(wid == 0, wid == NTILES // B))
    def _():
        lax.fori_loop(0, HW // 16, zero_body, 0, unroll=8)
        lax.fori_loop(0, N // 16, hit_body, 0, unroll=8)
        pltpu.sync_copy(acc_v, hits_hbm.at[pl.ds(b * HW, HW)])


def _run_sc_scatter(ot_flat, lin_flat):
    fn = pl.kernel(
        _sc_scatter_body,
        out_type=(
            jax.ShapeDtypeStruct((ROWS * HW,), jnp.float32),
            jax.ShapeDtypeStruct((B * HW,), jnp.float32),
        ),
        mesh=_SC_MESH,
        compiler_params=_SC_PARAMS,
        scratch_types=[
            pltpu.VMEM((HW,), jnp.float32),
            pltpu.VMEM((N,), jnp.int32),
            pltpu.VMEM((N,), jnp.float32),
        ],
    )
    return fn(ot_flat, lin_flat)


def _head_onehot():
    col = lax.broadcasted_iota(jnp.int32, (HEADS, HD), 1) // DHEAD
    row = lax.broadcasted_iota(jnp.int32, (HEADS, HD), 0)
    return (col == row).astype(jnp.float32)


def _tc1_body(g_ref, tok_ref, wq_ref, bq_ref, wk_ref, s_ref):
    g = g_ref[0]
    tok = tok_ref[0]
    q = jnp.dot(wq_ref[...], g, preferred_element_type=jnp.float32) + bq_ref[...]
    k = lax.dot_general(wk_ref[...], tok, (((1,), (1,)), ((), ())),
                        preferred_element_type=jnp.float32)
    s = jnp.dot(_head_onehot(), q * k, preferred_element_type=jnp.float32)
    s_ref[0] = s * SCALE


def _tc1_call(g3, tok, Wq, bq2, Wk):
    return pl.pallas_call(
        _tc1_body,
        grid=(B, NSTEPS),
        in_specs=[
            pl.BlockSpec((1, C, NC), lambda b, n: (b, 0, n)),
            pl.BlockSpec((1, NC, C_CAM), lambda b, n: (b, n, 0)),
            pl.BlockSpec((HD, C), lambda b, n: (0, 0)),
            pl.BlockSpec((HD, 1), lambda b, n: (0, 0)),
            pl.BlockSpec((HD, C_CAM), lambda b, n: (0, 0)),
        ],
        out_specs=pl.BlockSpec((1, HEADS, NC), lambda b, n: (b, 0, n)),
        out_shape=jax.ShapeDtypeStruct((B, HEADS, N), jnp.float32),
    )(g3, tok, Wq, bq2, Wk)


def _softmax_body(s_ref, gw_ref, p_ref):
    s = s_ref[0]
    m = jnp.max(s, axis=-1, keepdims=True)
    e = jnp.exp(s - m)
    z = jnp.sum(e, axis=-1, keepdims=True)
    p_ref[0] = e / z * gw_ref[0]


def _softmax_call(s3, gw3):
    return pl.pallas_call(
        _softmax_body,
        grid=(B,),
        in_specs=[
            pl.BlockSpec((1, HEADS, N), lambda b: (b, 0, 0)),
            pl.BlockSpec((1, 1, N), lambda b: (b, 0, 0)),
        ],
        out_specs=pl.BlockSpec((1, HEADS, N), lambda b: (b, 0, 0)),
        out_shape=jax.ShapeDtypeStruct((B, HEADS, N), jnp.float32),
    )(s3, gw3)


def _tc2_body(tok_ref, p_ref, wv_ref, wo_ref, ot_ref, cs_ref):
    nstep = pl.program_id(1)
    tok = tok_ref[0]
    v = lax.dot_general(wv_ref[...], tok, (((1,), (1,)), ((), ())),
                        preferred_element_type=jnp.float32)
    p = p_ref[0]
    pe = lax.dot_general(_head_onehot(), p, (((0,), (0,)), ((), ())),
                         preferred_element_type=jnp.float32)
    lane = lax.broadcasted_iota(jnp.int32, (HD, NC), 1) + nstep * NC
    fused = jnp.where(lane < N, pe * v, 0.0)
    ot_ref[0] = jnp.dot(wo_ref[...], fused, preferred_element_type=jnp.float32)
    cs = jnp.dot(wo_ref[...], jnp.sum(fused, axis=1, keepdims=True),
                 preferred_element_type=jnp.float32)

    @pl.when(nstep == 0)
    def _():
        cs_ref[0] = cs

    @pl.when(nstep > 0)
    def _():
        cs_ref[0] += cs


def _tc2_call(tok, p3, Wv, Wo):
    return pl.pallas_call(
        _tc2_body,
        grid=(B, NSTEPS),
        in_specs=[
            pl.BlockSpec((1, NC, C_CAM), lambda b, n: (b, n, 0)),
            pl.BlockSpec((1, HEADS, NC), lambda b, n: (b, 0, n)),
            pl.BlockSpec((HD, C_CAM), lambda b, n: (0, 0)),
            pl.BlockSpec((C, HD), lambda b, n: (0, 0)),
        ],
        out_specs=[
            pl.BlockSpec((1, C, NC), lambda b, n: (b, 0, n)),
            pl.BlockSpec((1, C, 1), lambda b, n: (b, 0, 0)),
        ],
        out_shape=[
            jax.ShapeDtypeStruct((B, C, N), jnp.float32),
            jax.ShapeDtypeStruct((B, C, 1), jnp.float32),
        ],
    )(tok, p3, Wv, Wo)


def _combine_body(lid_ref, dl_ref, al_ref, ht_ref, hf_ref, cs_ref, o_ref):
    hits_full = hf_ref[0]
    nhit = jnp.sum((hits_full > 0.0).astype(jnp.float32))
    mean = cs_ref[0] / (nhit + 1e-6)
    mask = (ht_ref[0] > 0.0).astype(jnp.float32)
    d = dl_ref[0] - mean * mask
    o_ref[0] = lid_ref[0] + d * (al_ref[0] * GAMMA)


def _combine_call(lidar3, delta3, alpha3, hits3, cs3):
    return pl.pallas_call(
        _combine_body,
        grid=(B, HW // PC),
        in_specs=[
            pl.BlockSpec((1, C, PC), lambda b, p: (b, 0, p)),
            pl.BlockSpec((1, C, PC), lambda b, p: (b, 0, p)),
            pl.BlockSpec((1, 1, PC), lambda b, p: (0, 0, p)),
            pl.BlockSpec((1, 1, PC), lambda b, p: (b, 0, p)),
            pl.BlockSpec((1, 1, HW), lambda b, p: (b, 0, 0)),
            pl.BlockSpec((1, C, 1), lambda b, p: (b, 0, 0)),
        ],
        out_specs=pl.BlockSpec((1, C, PC), lambda b, p: (b, 0, p)),
        out_shape=jax.ShapeDtypeStruct((B, C, HW), jnp.float32),
    )(lidar3, delta3, alpha3, hits3, hits3, cs3)


def kernel(lidar_bev, cam_bev_tokens, cam_bev_indices, gate_weights,
           range_alpha, Wq, bq, Wk, Wv, Wo):
    lidar3 = lidar_bev.reshape(B, C, HW)
    lidar_flat = lidar_bev.reshape(B * C * HW)
    ind = cam_bev_indices.astype(jnp.int32)
    ii3 = ind[..., 0].reshape(B, 1, N)
    jj3 = ind[..., 1].reshape(B, 1, N)
    gw3 = gate_weights.reshape(B, 1, N)
    alpha3 = range_alpha.reshape(1, 1, HW)
    bq2 = bq.reshape(HD, 1)

    lin_flat = _lin_call(ii3, jj3).reshape(B * N)
    g3 = _run_sc_gather(lidar_flat, lin_flat).reshape(B, C, N)
    s3 = _tc1_call(g3, cam_bev_tokens, Wq, bq2, Wk)
    p3 = _softmax_call(s3, gw3)
    ot3, cs3 = _tc2_call(cam_bev_tokens, p3, Wv, Wo)
    delta_flat, hits_flat = _run_sc_scatter(ot3.reshape(B * C * N), lin_flat)
    out3 = _combine_call(lidar3, delta_flat.reshape(B, C, HW), alpha3,
                         hits_flat.reshape(B, 1, HW), cs3)
    return out3.reshape(B, C, H, W)


# speedup vs baseline: 8.3069x; 1.2936x over previous
import functools
import math

import jax
import jax.numpy as jnp
from jax import lax
from jax.experimental import pallas as pl
from jax.experimental.pallas import tpu as pltpu
from jax.experimental.pallas import tpu_sc as plsc

B, C, H, W = 2, 128, 256, 256
HW = H * W
N, C_CAM = 20000, 128
HEADS, DHEAD = 4, 32
HD = HEADS * DHEAD
GAMMA = 0.08
SCALE = 1.0 / math.sqrt(DHEAD)

NTILES = 32
ROWS = B * C
ROWS_PER_TILE = ROWS // NTILES
NC = 2048
NSTEPS = (N + NC - 1) // NC
PC = 2048

_SC_MESH = plsc.VectorSubcoreMesh(core_axis_name="c", subcore_axis_name="s")
_SC_PARAMS = pltpu.CompilerParams(needs_layout_passes=False)


def _lin_body(ii_ref, jj_ref, lin_ref):
    i = jnp.clip(ii_ref[...], 0, H - 1)
    j = jnp.clip(jj_ref[...], 0, W - 1)
    lin_ref[...] = ((i // 8) * (W // 128) + j // 128) * 1024 \
        + (i % 8) * 128 + (j % 128)


def _lin_call(ii3, jj3):
    return pl.pallas_call(
        _lin_body,
        grid=(B,),
        in_specs=[
            pl.BlockSpec((1, 1, N), lambda b: (b, 0, 0)),
            pl.BlockSpec((1, 1, N), lambda b: (b, 0, 0)),
        ],
        out_specs=pl.BlockSpec((1, 1, N), lambda b: (b, 0, 0)),
        out_shape=jax.ShapeDtypeStruct((B, 1, N), jnp.int32),
    )(ii3, jj3)


def _sc_gather_body(lidar_hbm, lin_hbm, g_hbm, row_v, idx_v, out_v):
    cid = lax.axis_index("c")
    sid = lax.axis_index("s")
    wid = sid * 2 + cid
    b = wid // (NTILES // B)
    pltpu.sync_copy(lin_hbm.at[pl.ds(b * N, N)], idx_v)

    def body(i, _):
        off = i * 16
        ix = idx_v[pl.ds(off, 16)]
        out_v[pl.ds(off, 16)] = plsc.load_gather(row_v, [ix])
        return 0

    for k in range(ROWS_PER_TILE):
        r = wid * ROWS_PER_TILE + k
        pltpu.sync_copy(lidar_hbm.at[pl.ds(r * HW, HW)], row_v)
        lax.fori_loop(0, N // 16, body, 0, unroll=8)
        pltpu.sync_copy(out_v, g_hbm.at[pl.ds(r * N, N)])


def _run_sc_gather(lidar_flat, lin_flat):
    fn = pl.kernel(
        _sc_gather_body,
        out_type=jax.ShapeDtypeStruct((ROWS * N,), jnp.float32),
        mesh=_SC_MESH,
        compiler_params=_SC_PARAMS,
        scratch_types=[
            pltpu.VMEM((HW,), jnp.float32),
            pltpu.VMEM((N,), jnp.int32),
            pltpu.VMEM((N,), jnp.float32),
        ],
    )
    return fn(lidar_flat, lin_flat)


def _sc_scatter_body(ot_hbm, lin_hbm, delta_hbm, hits_hbm, acc_v, idx_v, dat_v):
    cid = lax.axis_index("c")
    sid = lax.axis_index("s")
    wid = sid * 2 + cid
    b = wid // (NTILES // B)
    pltpu.sync_copy(lin_hbm.at[pl.ds(b * N, N)], idx_v)
    zero16 = jnp.zeros((16,), jnp.float32)
    ones16 = jnp.ones((16,), jnp.float32)

    def zero_body(i, _):
        acc_v[pl.ds(i * 16, 16)] = zero16
        return 0

    def add_body(i, _):
        off = i * 16
        ix = idx_v[pl.ds(off, 16)]
        d = dat_v[pl.ds(off, 16)]
        plsc.addupdate_scatter(acc_v, [ix], d)
        return 0

    def hit_body(i, _):
        ix = idx_v[pl.ds(i * 16, 16)]
        plsc.addupdate_scatter(acc_v, [ix], ones16)
        return 0

    def unhit_body(i, _):
        ix = idx_v[pl.ds(i * 16, 16)]
        plsc.store_scatter(acc_v, [ix], zero16)
        return 0

    lax.fori_loop(0, HW // 16, zero_body, 0, unroll=8)
    for k in range(ROWS_PER_TILE):
        r = wid * ROWS_PER_TILE + k
        pltpu.sync_copy(ot_hbm.at[pl.ds(r * N, N)], dat_v)
        lax.fori_loop(0, N // 16, add_body, 0, unroll=8)
        pltpu.sync_copy(acc_v, delta_hbm.at[pl.ds(r * HW, HW)])
        lax.fori_loop(0, N // 16, unhit_body, 0, unroll=8)

    @pl.when(jnp.logical_or(wid == 0, wid == NTILES // B))
    def _():
        lax.fori_loop(0, N // 16, hit_body, 0, unroll=8)
        pltpu.sync_copy(acc_v, hits_hbm.at[pl.ds(b * HW, HW)])


def _run_sc_scatter(ot_flat, lin_flat):
    fn = pl.kernel(
        _sc_scatter_body,
        out_type=(
            jax.ShapeDtypeStruct((ROWS * HW,), jnp.float32),
            jax.ShapeDtypeStruct((B * HW,), jnp.float32),
        ),
        mesh=_SC_MESH,
        compiler_params=_SC_PARAMS,
        scratch_types=[
            pltpu.VMEM((HW,), jnp.float32),
            pltpu.VMEM((N,), jnp.int32),
            pltpu.VMEM((N,), jnp.float32),
        ],
    )
    return fn(ot_flat, lin_flat)


def _head_onehot():
    col = lax.broadcasted_iota(jnp.int32, (HEADS, HD), 1) // DHEAD
    row = lax.broadcasted_iota(jnp.int32, (HEADS, HD), 0)
    return (col == row).astype(jnp.float32)


def _tc1_body(g_ref, tok_ref, wq_ref, bq_ref, wk_ref, s_ref):
    g = g_ref[0]
    tok = tok_ref[0]
    q = jnp.dot(wq_ref[...], g, preferred_element_type=jnp.float32) + bq_ref[...]
    k = lax.dot_general(wk_ref[...], tok, (((1,), (1,)), ((), ())),
                        preferred_element_type=jnp.float32)
    s = jnp.dot(_head_onehot(), q * k, preferred_element_type=jnp.float32)
    s_ref[0] = s * SCALE


def _tc1_call(g3, tok, Wq, bq2, Wk):
    return pl.pallas_call(
        _tc1_body,
        grid=(B, NSTEPS),
        in_specs=[
            pl.BlockSpec((1, C, NC), lambda b, n: (b, 0, n)),
            pl.BlockSpec((1, NC, C_CAM), lambda b, n: (b, n, 0)),
            pl.BlockSpec((HD, C), lambda b, n: (0, 0)),
            pl.BlockSpec((HD, 1), lambda b, n: (0, 0)),
            pl.BlockSpec((HD, C_CAM), lambda b, n: (0, 0)),
        ],
        out_specs=pl.BlockSpec((1, HEADS, NC), lambda b, n: (b, 0, n)),
        out_shape=jax.ShapeDtypeStruct((B, HEADS, N), jnp.float32),
    )(g3, tok, Wq, bq2, Wk)


def _softmax_body(s_ref, gw_ref, p_ref):
    s = s_ref[0]
    m = jnp.max(s, axis=-1, keepdims=True)
    e = jnp.exp(s - m)
    z = jnp.sum(e, axis=-1, keepdims=True)
    p_ref[0] = e / z * gw_ref[0]


def _softmax_call(s3, gw3):
    return pl.pallas_call(
        _softmax_body,
        grid=(B,),
        in_specs=[
            pl.BlockSpec((1, HEADS, N), lambda b: (b, 0, 0)),
            pl.BlockSpec((1, 1, N), lambda b: (b, 0, 0)),
        ],
        out_specs=pl.BlockSpec((1, HEADS, N), lambda b: (b, 0, 0)),
        out_shape=jax.ShapeDtypeStruct((B, HEADS, N), jnp.float32),
    )(s3, gw3)


def _tc2_body(tok_ref, p_ref, wv_ref, wo_ref, ot_ref, cs_ref):
    nstep = pl.program_id(1)
    tok = tok_ref[0]
    v = lax.dot_general(wv_ref[...], tok, (((1,), (1,)), ((), ())),
                        preferred_element_type=jnp.float32)
    p = p_ref[0]
    pe = lax.dot_general(_head_onehot(), p, (((0,), (0,)), ((), ())),
                         preferred_element_type=jnp.float32)
    lane = lax.broadcasted_iota(jnp.int32, (HD, NC), 1) + nstep * NC
    fused = jnp.where(lane < N, pe * v, 0.0)
    ot_ref[0] = jnp.dot(wo_ref[...], fused, preferred_element_type=jnp.float32)
    cs = jnp.dot(wo_ref[...], jnp.sum(fused, axis=1, keepdims=True),
                 preferred_element_type=jnp.float32)

    @pl.when(nstep == 0)
    def _():
        cs_ref[0] = cs

    @pl.when(nstep > 0)
    def _():
        cs_ref[0] += cs


def _tc2_call(tok, p3, Wv, Wo):
    return pl.pallas_call(
        _tc2_body,
        grid=(B, NSTEPS),
        in_specs=[
            pl.BlockSpec((1, NC, C_CAM), lambda b, n: (b, n, 0)),
            pl.BlockSpec((1, HEADS, NC), lambda b, n: (b, 0, n)),
            pl.BlockSpec((HD, C_CAM), lambda b, n: (0, 0)),
            pl.BlockSpec((C, HD), lambda b, n: (0, 0)),
        ],
        out_specs=[
            pl.BlockSpec((1, C, NC), lambda b, n: (b, 0, n)),
            pl.BlockSpec((1, C, 1), lambda b, n: (b, 0, 0)),
        ],
        out_shape=[
            jax.ShapeDtypeStruct((B, C, N), jnp.float32),
            jax.ShapeDtypeStruct((B, C, 1), jnp.float32),
        ],
    )(tok, p3, Wv, Wo)


HG = H // 8
WG = W // 128


def _tiles_to_pixels(x):
    return jnp.concatenate([x[..., g, :, :] for g in range(WG)], axis=-1)


def _combine_body(lid_ref, dl_ref, al_ref, ht_ref, hf_ref, cs_ref, o_ref):
    hits_full = hf_ref[0]
    nhit = jnp.sum((hits_full > 0.0).astype(jnp.float32))
    mean = cs_ref[0].reshape(C, 1, 1) / (nhit + 1e-6)
    d = _tiles_to_pixels(dl_ref[0, :, 0])
    mask = (_tiles_to_pixels(ht_ref[0, 0]) > 0.0).astype(jnp.float32)
    dd = d - mean * mask[None]
    o_ref[0] = lid_ref[0] + dd * (al_ref[0] * GAMMA)


def _combine_call(lidar4, delta6, alpha4, hits5, cs3):
    return pl.pallas_call(
        _combine_body,
        grid=(B, HG),
        in_specs=[
            pl.BlockSpec((1, C, 8, W), lambda b, p: (b, 0, p, 0)),
            pl.BlockSpec((1, C, 1, WG, 8, 128), lambda b, p: (b, 0, p, 0, 0, 0)),
            pl.BlockSpec((1, 1, 8, W), lambda b, p: (0, 0, p, 0)),
            pl.BlockSpec((1, 1, WG, 8, 128), lambda b, p: (b, p, 0, 0, 0)),
            pl.BlockSpec((1, HG, WG, 8, 128), lambda b, p: (b, 0, 0, 0, 0)),
            pl.BlockSpec((1, C, 1), lambda b, p: (b, 0, 0)),
        ],
        out_specs=pl.BlockSpec((1, C, 8, W), lambda b, p: (b, 0, p, 0)),
        out_shape=jax.ShapeDtypeStruct((B, C, H, W), jnp.float32),
    )(lidar4, delta6, alpha4, hits5, hits5, cs3)


def kernel(lidar_bev, cam_bev_tokens, cam_bev_indices, gate_weights,
           range_alpha, Wq, bq, Wk, Wv, Wo):
    lidar_phys = lidar_bev.reshape(B, C, HG, 8, WG, 128) \
        .transpose(0, 1, 2, 4, 3, 5).reshape(B * C * HW)
    ind = cam_bev_indices.astype(jnp.int32)
    ii3 = ind[..., 0].reshape(B, 1, N)
    jj3 = ind[..., 1].reshape(B, 1, N)
    gw3 = gate_weights.reshape(B, 1, N)
    alpha4 = range_alpha
    bq2 = bq.reshape(HD, 1)

    lin_flat = _lin_call(ii3, jj3).reshape(B * N)
    g3 = _run_sc_gather(lidar_phys, lin_flat).reshape(B, C, N)
    s3 = _tc1_call(g3, cam_bev_tokens, Wq, bq2, Wk)
    p3 = _softmax_call(s3, gw3)
    ot3, cs3 = _tc2_call(cam_bev_tokens, p3, Wv, Wo)
    delta_flat, hits_flat = _run_sc_scatter(ot3.reshape(B * C * N), lin_flat)
    delta6 = delta_flat.reshape(B, C, HG, WG, 8, 128)
    hits5 = hits_flat.reshape(B, HG, WG, 8, 128)
    return _combine_call(lidar_bev, delta6, alpha4, hits5, cs3)


# speedup vs baseline: 12.3485x; 1.4865x over previous
import functools
import math

import jax
import jax.numpy as jnp
from jax import lax
from jax.experimental import pallas as pl
from jax.experimental.pallas import tpu as pltpu
from jax.experimental.pallas import tpu_sc as plsc

B, C, H, W = 2, 128, 256, 256
HW = H * W
N, C_CAM = 20000, 128
HEADS, DHEAD = 4, 32
HD = HEADS * DHEAD
GAMMA = 0.08
SCALE = 1.0 / math.sqrt(DHEAD)

NTILES = 32
ROWS = B * C
ROWS_PER_TILE = ROWS // NTILES
NC = 2048
NSTEPS = (N + NC - 1) // NC
PC = 2048

_SC_MESH = plsc.VectorSubcoreMesh(core_axis_name="c", subcore_axis_name="s")
_SC_PARAMS = pltpu.CompilerParams(needs_layout_passes=False)


def _lin_body(ii_ref, jj_ref, lin_ref):
    i = jnp.clip(ii_ref[...], 0, H - 1)
    j = jnp.clip(jj_ref[...], 0, W - 1)
    lin_ref[...] = ((i // 8) * (W // 128) + j // 128) * 1024 \
        + (i % 8) * 128 + (j % 128)


def _lin_call(ii3, jj3):
    return pl.pallas_call(
        _lin_body,
        grid=(B,),
        in_specs=[
            pl.BlockSpec((1, 1, N), lambda b: (b, 0, 0)),
            pl.BlockSpec((1, 1, N), lambda b: (b, 0, 0)),
        ],
        out_specs=pl.BlockSpec((1, 1, N), lambda b: (b, 0, 0)),
        out_shape=jax.ShapeDtypeStruct((B, 1, N), jnp.int32),
    )(ii3, jj3)


def _sc_gather_body(lidar_hbm, lin_hbm, g_hbm, row_v, idx_v, out_v):
    cid = lax.axis_index("c")
    sid = lax.axis_index("s")
    wid = sid * 2 + cid
    b = wid // (NTILES // B)
    pltpu.sync_copy(lin_hbm.at[pl.ds(b * N, N)], idx_v)

    for k in range(ROWS_PER_TILE):
        r = wid * ROWS_PER_TILE + k
        pltpu.sync_copy(lidar_hbm.at[pl.ds(r * HW, HW)], row_v)

        @plsc.parallel_loop(0, N // 16, unroll=8)
        def _(i):
            off = i * 16
            ix = idx_v[pl.ds(off, 16)]
            out_v[pl.ds(off, 16)] = plsc.load_gather(row_v, [ix])

        pltpu.sync_copy(out_v, g_hbm.at[pl.ds(r * N, N)])


def _run_sc_gather(lidar_flat, lin_flat):
    fn = pl.kernel(
        _sc_gather_body,
        out_type=jax.ShapeDtypeStruct((ROWS * N,), jnp.float32),
        mesh=_SC_MESH,
        compiler_params=_SC_PARAMS,
        scratch_types=[
            pltpu.VMEM((HW,), jnp.float32),
            pltpu.VMEM((N,), jnp.int32),
            pltpu.VMEM((N,), jnp.float32),
        ],
    )
    return fn(lidar_flat, lin_flat)


def _sc_scatter_body(ot_hbm, lin_hbm, delta_hbm, hits_hbm, acc_v, idx_v, dat_v):
    cid = lax.axis_index("c")
    sid = lax.axis_index("s")
    wid = sid * 2 + cid
    b = wid // (NTILES // B)
    pltpu.sync_copy(lin_hbm.at[pl.ds(b * N, N)], idx_v)
    zero16 = jnp.zeros((16,), jnp.float32)
    ones16 = jnp.ones((16,), jnp.float32)

    def scatter_add_loop():
        @plsc.parallel_loop(0, N // 16, unroll=8)
        def _(i):
            off = i * 16
            ix = idx_v[pl.ds(off, 16)]
            d = dat_v[pl.ds(off, 16)]
            plsc.addupdate_scatter(acc_v, [ix], d)

    def scatter_zero_loop():
        @plsc.parallel_loop(0, N // 16, unroll=8)
        def _(i):
            ix = idx_v[pl.ds(i * 16, 16)]
            plsc.store_scatter(acc_v, [ix], zero16)

    @plsc.parallel_loop(0, HW // 16, unroll=8)
    def _(i):
        acc_v[pl.ds(i * 16, 16)] = zero16

    for k in range(ROWS_PER_TILE):
        r = wid * ROWS_PER_TILE + k
        pltpu.sync_copy(ot_hbm.at[pl.ds(r * N, N)], dat_v)
        scatter_add_loop()
        pltpu.sync_copy(acc_v, delta_hbm.at[pl.ds(r * HW, HW)])
        scatter_zero_loop()

    @pl.when(jnp.logical_or(wid == 0, wid == NTILES // B))
    def _():
        @plsc.parallel_loop(0, N // 16, unroll=8)
        def _(i):
            ix = idx_v[pl.ds(i * 16, 16)]
            plsc.addupdate_scatter(acc_v, [ix], ones16)

        pltpu.sync_copy(acc_v, hits_hbm.at[pl.ds(b * HW, HW)])


def _run_sc_scatter(ot_flat, lin_flat):
    fn = pl.kernel(
        _sc_scatter_body,
        out_type=(
            jax.ShapeDtypeStruct((ROWS * HW,), jnp.float32),
            jax.ShapeDtypeStruct((B * HW,), jnp.float32),
        ),
        mesh=_SC_MESH,
        compiler_params=_SC_PARAMS,
        scratch_types=[
            pltpu.VMEM((HW,), jnp.float32),
            pltpu.VMEM((N,), jnp.int32),
            pltpu.VMEM((N,), jnp.float32),
        ],
    )
    return fn(ot_flat, lin_flat)


def _head_onehot():
    col = lax.broadcasted_iota(jnp.int32, (HEADS, HD), 1) // DHEAD
    row = lax.broadcasted_iota(jnp.int32, (HEADS, HD), 0)
    return (col == row).astype(jnp.float32)


def _tc1_body(g_ref, tok_ref, wq_ref, bq_ref, wk_ref, s_ref):
    g = g_ref[0]
    tok = tok_ref[0]
    q = jnp.dot(wq_ref[...], g, preferred_element_type=jnp.float32) + bq_ref[...]
    k = lax.dot_general(wk_ref[...], tok, (((1,), (1,)), ((), ())),
                        preferred_element_type=jnp.float32)
    s = jnp.dot(_head_onehot(), q * k, preferred_element_type=jnp.float32)
    s_ref[0] = s * SCALE


def _tc1_call(g3, tok, Wq, bq2, Wk):
    return pl.pallas_call(
        _tc1_body,
        grid=(B, NSTEPS),
        in_specs=[
            pl.BlockSpec((1, C, NC), lambda b, n: (b, 0, n)),
            pl.BlockSpec((1, NC, C_CAM), lambda b, n: (b, n, 0)),
            pl.BlockSpec((HD, C), lambda b, n: (0, 0)),
            pl.BlockSpec((HD, 1), lambda b, n: (0, 0)),
            pl.BlockSpec((HD, C_CAM), lambda b, n: (0, 0)),
        ],
        out_specs=pl.BlockSpec((1, HEADS, NC), lambda b, n: (b, 0, n)),
        out_shape=jax.ShapeDtypeStruct((B, HEADS, N), jnp.float32),
    )(g3, tok, Wq, bq2, Wk)


def _softmax_body(s_ref, gw_ref, p_ref):
    s = s_ref[0]
    m = jnp.max(s, axis=-1, keepdims=True)
    e = jnp.exp(s - m)
    z = jnp.sum(e, axis=-1, keepdims=True)
    p_ref[0] = e / z * gw_ref[0]


def _softmax_call(s3, gw3):
    return pl.pallas_call(
        _softmax_body,
        grid=(B,),
        in_specs=[
            pl.BlockSpec((1, HEADS, N), lambda b: (b, 0, 0)),
            pl.BlockSpec((1, 1, N), lambda b: (b, 0, 0)),
        ],
        out_specs=pl.BlockSpec((1, HEADS, N), lambda b: (b, 0, 0)),
        out_shape=jax.ShapeDtypeStruct((B, HEADS, N), jnp.float32),
    )(s3, gw3)


def _tc2_body(tok_ref, p_ref, wv_ref, wo_ref, ot_ref, cs_ref):
    nstep = pl.program_id(1)
    tok = tok_ref[0]
    v = lax.dot_general(wv_ref[...], tok, (((1,), (1,)), ((), ())),
                        preferred_element_type=jnp.float32)
    p = p_ref[0]
    pe = lax.dot_general(_head_onehot(), p, (((0,), (0,)), ((), ())),
                         preferred_element_type=jnp.float32)
    lane = lax.broadcasted_iota(jnp.int32, (HD, NC), 1) + nstep * NC
    fused = jnp.where(lane < N, pe * v, 0.0)
    ot_ref[0] = jnp.dot(wo_ref[...], fused, preferred_element_type=jnp.float32)
    cs = jnp.dot(wo_ref[...], jnp.sum(fused, axis=1, keepdims=True),
                 preferred_element_type=jnp.float32)

    @pl.when(nstep == 0)
    def _():
        cs_ref[0] = cs

    @pl.when(nstep > 0)
    def _():
        cs_ref[0] += cs


def _tc2_call(tok, p3, Wv, Wo):
    return pl.pallas_call(
        _tc2_body,
        grid=(B, NSTEPS),
        in_specs=[
            pl.BlockSpec((1, NC, C_CAM), lambda b, n: (b, n, 0)),
            pl.BlockSpec((1, HEADS, NC), lambda b, n: (b, 0, n)),
            pl.BlockSpec((HD, C_CAM), lambda b, n: (0, 0)),
            pl.BlockSpec((C, HD), lambda b, n: (0, 0)),
        ],
        out_specs=[
            pl.BlockSpec((1, C, NC), lambda b, n: (b, 0, n)),
            pl.BlockSpec((1, C, 1), lambda b, n: (b, 0, 0)),
        ],
        out_shape=[
            jax.ShapeDtypeStruct((B, C, N), jnp.float32),
            jax.ShapeDtypeStruct((B, C, 1), jnp.float32),
        ],
    )(tok, p3, Wv, Wo)


HG = H // 8
WG = W // 128


def _tiles_to_pixels(x):
    return jnp.concatenate([x[..., g, :, :] for g in range(WG)], axis=-1)


def _combine_body(lid_ref, dl_ref, al_ref, ht_ref, hf_ref, cs_ref, o_ref):
    hits_full = hf_ref[0]
    nhit = jnp.sum((hits_full > 0.0).astype(jnp.float32))
    mean = cs_ref[0].reshape(C, 1, 1) / (nhit + 1e-6)
    d = _tiles_to_pixels(dl_ref[0, :, 0])
    mask = (_tiles_to_pixels(ht_ref[0, 0]) > 0.0).astype(jnp.float32)
    dd = d - mean * mask[None]
    o_ref[0] = lid_ref[0] + dd * (al_ref[0] * GAMMA)


def _combine_call(lidar4, delta6, alpha4, hits5, cs3):
    return pl.pallas_call(
        _combine_body,
        grid=(B, HG),
        in_specs=[
            pl.BlockSpec((1, C, 8, W), lambda b, p: (b, 0, p, 0)),
            pl.BlockSpec((1, C, 1, WG, 8, 128), lambda b, p: (b, 0, p, 0, 0, 0)),
            pl.BlockSpec((1, 1, 8, W), lambda b, p: (0, 0, p, 0)),
            pl.BlockSpec((1, 1, WG, 8, 128), lambda b, p: (b, p, 0, 0, 0)),
            pl.BlockSpec((1, HG, WG, 8, 128), lambda b, p: (b, 0, 0, 0, 0)),
            pl.BlockSpec((1, C, 1), lambda b, p: (b, 0, 0)),
        ],
        out_specs=pl.BlockSpec((1, C, 8, W), lambda b, p: (b, 0, p, 0)),
        out_shape=jax.ShapeDtypeStruct((B, C, H, W), jnp.float32),
    )(lidar4, delta6, alpha4, hits5, hits5, cs3)


def kernel(lidar_bev, cam_bev_tokens, cam_bev_indices, gate_weights,
           range_alpha, Wq, bq, Wk, Wv, Wo):
    lidar_phys = lidar_bev.reshape(B, C, HG, 8, WG, 128) \
        .transpose(0, 1, 2, 4, 3, 5).reshape(B * C * HW)
    ind = cam_bev_indices.astype(jnp.int32)
    ii3 = ind[..., 0].reshape(B, 1, N)
    jj3 = ind[..., 1].reshape(B, 1, N)
    gw3 = gate_weights.reshape(B, 1, N)
    alpha4 = range_alpha
    bq2 = bq.reshape(HD, 1)

    lin_flat = _lin_call(ii3, jj3).reshape(B * N)
    g3 = _run_sc_gather(lidar_phys, lin_flat).reshape(B, C, N)
    s3 = _tc1_call(g3, cam_bev_tokens, Wq, bq2, Wk)
    p3 = _softmax_call(s3, gw3)
    ot3, cs3 = _tc2_call(cam_bev_tokens, p3, Wv, Wo)
    delta_flat, hits_flat = _run_sc_scatter(ot3.reshape(B * C * N), lin_flat)
    delta6 = delta_flat.reshape(B, C, HG, WG, 8, 128)
    hits5 = hits_flat.reshape(B, HG, WG, 8, 128)
    return _combine_call(lidar_bev, delta6, alpha4, hits5, cs3)


# speedup vs baseline: 13.4070x; 1.0857x over previous
import functools
import math

import jax
import jax.numpy as jnp
from jax import lax
from jax.experimental import pallas as pl
from jax.experimental.pallas import tpu as pltpu
from jax.experimental.pallas import tpu_sc as plsc

B, C, H, W = 2, 128, 256, 256
HW = H * W
N, C_CAM = 20000, 128
HEADS, DHEAD = 4, 32
HD = HEADS * DHEAD
GAMMA = 0.08
SCALE = 1.0 / math.sqrt(DHEAD)

NTILES = 32
RPT = C // NTILES
NC = 2048
NSTEPS = (N + NC - 1) // NC
NP = NSTEPS * NC
HG = H // 8
WG = W // 128

_SC_MESH = plsc.VectorSubcoreMesh(core_axis_name="c", subcore_axis_name="s")
_SC_PARAMS = pltpu.CompilerParams(needs_layout_passes=False)


def _lin_body(ii_ref, jj_ref, lin_ref):
    i = jnp.clip(ii_ref[...], 0, H - 1)
    j = jnp.clip(jj_ref[...], 0, W - 1)
    lin_ref[...] = ((i // 8) * WG + j // 128) * 1024 \
        + (i % 8) * 128 + (j % 128)


def _lin_call(ii3, jj3):
    return pl.pallas_call(
        _lin_body,
        grid=(B,),
        in_specs=[
            pl.BlockSpec((1, 1, N), lambda b: (b, 0, 0)),
            pl.BlockSpec((1, 1, N), lambda b: (b, 0, 0)),
        ],
        out_specs=pl.BlockSpec((1, 1, N), lambda b: (b, 0, 0)),
        out_shape=jax.ShapeDtypeStruct((B, 1, N), jnp.int32),
    )(ii3, jj3)


def _sc_gather_body(b, lidar_hbm, lin_hbm, g_hbm, row_v, idx_v, out_v):
    cid = lax.axis_index("c")
    sid = lax.axis_index("s")
    wid = sid * 2 + cid
    pltpu.sync_copy(lin_hbm.at[pl.ds(b * N, N)], idx_v)

    for k in range(RPT):
        r = wid * RPT + k
        pltpu.sync_copy(lidar_hbm.at[pl.ds((b * C + r) * HW, HW)], row_v)

        @plsc.parallel_loop(0, N // 16, unroll=8)
        def _(i):
            off = i * 16
            ix = idx_v[pl.ds(off, 16)]
            out_v[pl.ds(off, 16)] = plsc.load_gather(row_v, [ix])

        pltpu.sync_copy(out_v, g_hbm.at[pl.ds(r * N, N)])


def _run_sc_gather(b, lidar_phys, lin_flat):
    fn = pl.kernel(
        functools.partial(_sc_gather_body, b),
        out_type=jax.ShapeDtypeStruct((C * N,), jnp.float32),
        mesh=_SC_MESH,
        compiler_params=_SC_PARAMS,
        scratch_types=[
            pltpu.VMEM((HW,), jnp.float32),
            pltpu.VMEM((N,), jnp.int32),
            pltpu.VMEM((N,), jnp.float32),
        ],
    )
    return fn(lidar_phys, lin_flat)


def _sc_scatter_body(b, ot_hbm, lin_hbm, delta_hbm, hits_hbm,
                     acc_v, idx_v, dat_v):
    cid = lax.axis_index("c")
    sid = lax.axis_index("s")
    wid = sid * 2 + cid
    pltpu.sync_copy(lin_hbm.at[pl.ds(b * N, N)], idx_v)
    zero16 = jnp.zeros((16,), jnp.float32)
    ones16 = jnp.ones((16,), jnp.float32)

    def scatter_add_loop():
        @plsc.parallel_loop(0, N // 16, unroll=8)
        def _(i):
            off = i * 16
            ix = idx_v[pl.ds(off, 16)]
            d = dat_v[pl.ds(off, 16)]
            plsc.addupdate_scatter(acc_v, [ix], d)

    def scatter_zero_loop():
        @plsc.parallel_loop(0, N // 16, unroll=8)
        def _(i):
            ix = idx_v[pl.ds(i * 16, 16)]
            plsc.store_scatter(acc_v, [ix], zero16)

    @plsc.parallel_loop(0, HW // 16, unroll=8)
    def _(i):
        acc_v[pl.ds(i * 16, 16)] = zero16

    for k in range(RPT):
        r = wid * RPT + k
        pltpu.sync_copy(ot_hbm.at[pl.ds(r * N, N)], dat_v)
        scatter_add_loop()
        pltpu.sync_copy(acc_v, delta_hbm.at[pl.ds(r * HW, HW)])
        scatter_zero_loop()

    @pl.when(wid == 0)
    def _():
        @plsc.parallel_loop(0, N // 16, unroll=8)
        def _(i):
            ix = idx_v[pl.ds(i * 16, 16)]
            plsc.addupdate_scatter(acc_v, [ix], ones16)

        pltpu.sync_copy(acc_v, hits_hbm)


def _run_sc_scatter(b, ot_flat, lin_flat):
    fn = pl.kernel(
        functools.partial(_sc_scatter_body, b),
        out_type=(
            jax.ShapeDtypeStruct((C * HW,), jnp.float32),
            jax.ShapeDtypeStruct((HW,), jnp.float32),
        ),
        mesh=_SC_MESH,
        compiler_params=_SC_PARAMS,
        scratch_types=[
            pltpu.VMEM((HW,), jnp.float32),
            pltpu.VMEM((N,), jnp.int32),
            pltpu.VMEM((N,), jnp.float32),
        ],
    )
    return fn(ot_flat, lin_flat)


def _head_onehot():
    col = lax.broadcasted_iota(jnp.int32, (HEADS, HD), 1) // DHEAD
    row = lax.broadcasted_iota(jnp.int32, (HEADS, HD), 0)
    return (col == row).astype(jnp.float32)


def _tc1_body(g_ref, tok_ref, wq_ref, bq_ref, wk_ref, s_ref):
    g = g_ref[0]
    tok = tok_ref[0]
    q = jnp.dot(wq_ref[...], g, preferred_element_type=jnp.float32) + bq_ref[...]
    k = lax.dot_general(wk_ref[...], tok, (((1,), (1,)), ((), ())),
                        preferred_element_type=jnp.float32)
    s = jnp.dot(_head_onehot(), q * k, preferred_element_type=jnp.float32)
    s_ref[0] = s * SCALE


def _tc1_call(b, g3, tok, Wq, bq2, Wk):
    return pl.pallas_call(
        _tc1_body,
        grid=(NSTEPS,),
        in_specs=[
            pl.BlockSpec((1, C, NC), lambda n: (0, 0, n)),
            pl.BlockSpec((1, NC, C_CAM), lambda n, _b=b: (_b, n, 0)),
            pl.BlockSpec((HD, C), lambda n: (0, 0)),
            pl.BlockSpec((HD, 1), lambda n: (0, 0)),
            pl.BlockSpec((HD, C_CAM), lambda n: (0, 0)),
        ],
        out_specs=pl.BlockSpec((1, HEADS, NC), lambda n: (0, 0, n)),
        out_shape=jax.ShapeDtypeStruct((1, HEADS, NP), jnp.float32),
    )(g3, tok, Wq, bq2, Wk)


def _tc2_body(s_ref, gw_ref, tok_ref, wv_ref, wo_ref, ot_ref, cs_ref, mz_ref):
    nstep = pl.program_id(0)
    lane_full = lax.broadcasted_iota(jnp.int32, (HEADS, NP), 1)

    @pl.when(nstep == 0)
    def _():
        s = s_ref[0]
        sm = jnp.where(lane_full < N, s, -jnp.inf)
        m = jnp.max(sm, axis=-1, keepdims=True)
        e = jnp.where(lane_full < N, jnp.exp(sm - m), 0.0)
        z = jnp.sum(e, axis=-1, keepdims=True)
        mz_ref[0:HEADS, 0:1] = m
        mz_ref[0:HEADS, 1:2] = z

    m = mz_ref[0:HEADS, 0:1]
    z = mz_ref[0:HEADS, 1:2]
    s_blk = s_ref[0, :, pl.ds(nstep * NC, NC)]
    lane = lax.broadcasted_iota(jnp.int32, (HEADS, NC), 1) + nstep * NC
    p = jnp.where(lane < N, jnp.exp(s_blk - m) / z, 0.0) * gw_ref[0]

    tok = tok_ref[0]
    v = lax.dot_general(wv_ref[...], tok, (((1,), (1,)), ((), ())),
                        preferred_element_type=jnp.float32)
    pe = lax.dot_general(_head_onehot(), p, (((0,), (0,)), ((), ())),
                         preferred_element_type=jnp.float32)
    lane2 = lax.broadcasted_iota(jnp.int32, (HD, NC), 1) + nstep * NC
    fused = jnp.where(lane2 < N, pe * v, 0.0)
    ot_ref[0] = jnp.dot(wo_ref[...], fused, preferred_element_type=jnp.float32)
    cs = jnp.dot(wo_ref[...], jnp.sum(fused, axis=1, keepdims=True),
                 preferred_element_type=jnp.float32)

    @pl.when(nstep == 0)
    def _():
        cs_ref[0] = cs

    @pl.when(nstep > 0)
    def _():
        cs_ref[0] += cs


def _tc2_call(b, s3, tok, gw3, Wv, Wo):
    return pl.pallas_call(
        _tc2_body,
        grid=(NSTEPS,),
        in_specs=[
            pl.BlockSpec((1, HEADS, NP), lambda n: (0, 0, 0)),
            pl.BlockSpec((1, 1, NC), lambda n, _b=b: (_b, 0, n)),
            pl.BlockSpec((1, NC, C_CAM), lambda n, _b=b: (_b, n, 0)),
            pl.BlockSpec((HD, C_CAM), lambda n: (0, 0)),
            pl.BlockSpec((C, HD), lambda n: (0, 0)),
        ],
        out_specs=[
            pl.BlockSpec((1, C, NC), lambda n: (0, 0, n)),
            pl.BlockSpec((1, C, 1), lambda n: (0, 0, 0)),
        ],
        out_shape=[
            jax.ShapeDtypeStruct((1, C, N), jnp.float32),
            jax.ShapeDtypeStruct((1, C, 1), jnp.float32),
        ],
        scratch_shapes=[pltpu.VMEM((8, 128), jnp.float32)],
    )(s3, gw3, tok, Wv, Wo)


def _tiles_to_pixels(x):
    return jnp.concatenate([x[..., g, :, :] for g in range(WG)], axis=-1)


def _sel(b, x0, x1):
    return jnp.where(b == 0, x0, x1)


def _combine_body(lid_ref, dl0_ref, dl1_ref, al_ref, ht0_ref, ht1_ref,
                  hf0_ref, hf1_ref, cs0_ref, cs1_ref, o_ref):
    b = pl.program_id(0)
    hits_full = _sel(b, hf0_ref[...], hf1_ref[...])
    nhit = jnp.sum((hits_full > 0.0).astype(jnp.float32))
    cs = _sel(b, cs0_ref[0], cs1_ref[0])
    mean = cs.reshape(C, 1, 1) / (nhit + 1e-6)
    d = _tiles_to_pixels(_sel(b, dl0_ref[:, 0], dl1_ref[:, 0]))
    ht = _sel(b, ht0_ref[0], ht1_ref[0])
    mask = (_tiles_to_pixels(ht) > 0.0).astype(jnp.float32)
    dd = d - mean * mask[None]
    o_ref[0] = lid_ref[0] + dd * (al_ref[0] * GAMMA)


def _combine_call(lidar4, d0, d1, alpha4, h0, h1, cs0, cs1):
    z = lambda n: (0, 0, 0, 0)

    def dmap0(b, p):
        return (0, jnp.where(b == 0, p, 0), 0, 0, 0)

    def dmap1(b, p):
        return (0, jnp.where(b == 0, 0, p), 0, 0, 0)

    def hmap0(b, p):
        return (jnp.where(b == 0, p, 0), 0, 0, 0)

    def hmap1(b, p):
        return (jnp.where(b == 0, 0, p), 0, 0, 0)

    return pl.pallas_call(
        _combine_body,
        grid=(B, HG),
        in_specs=[
            pl.BlockSpec((1, C, 8, W), lambda b, p: (b, 0, p, 0)),
            pl.BlockSpec((C, 1, WG, 8, 128), dmap0),
            pl.BlockSpec((C, 1, WG, 8, 128), dmap1),
            pl.BlockSpec((1, 1, 8, W), lambda b, p: (0, 0, p, 0)),
            pl.BlockSpec((1, WG, 8, 128), hmap0),
            pl.BlockSpec((1, WG, 8, 128), hmap1),
            pl.BlockSpec((HG, WG, 8, 128), lambda b, p: (0, 0, 0, 0)),
            pl.BlockSpec((HG, WG, 8, 128), lambda b, p: (0, 0, 0, 0)),
            pl.BlockSpec((1, C, 1), lambda b, p: (0, 0, 0)),
            pl.BlockSpec((1, C, 1), lambda b, p: (0, 0, 0)),
        ],
        out_specs=pl.BlockSpec((1, C, 8, W), lambda b, p: (b, 0, p, 0)),
        out_shape=jax.ShapeDtypeStruct((B, C, H, W), jnp.float32),
    )(lidar4, d0, d1, alpha4, h0, h1, h0, h1, cs0, cs1)


def kernel(lidar_bev, cam_bev_tokens, cam_bev_indices, gate_weights,
           range_alpha, Wq, bq, Wk, Wv, Wo):
    lidar_phys = lidar_bev.reshape(B, C, HG, 8, WG, 128) \
        .transpose(0, 1, 2, 4, 3, 5).reshape(B * C * HW)
    ind = cam_bev_indices.astype(jnp.int32)
    ii3 = ind[..., 0].reshape(B, 1, N)
    jj3 = ind[..., 1].reshape(B, 1, N)
    gw3 = gate_weights.reshape(B, 1, N)
    bq2 = bq.reshape(HD, 1)

    lin_flat = _lin_call(ii3, jj3).reshape(B * N)

    d, h, cs = [], [], []
    for b in range(B):
        g3 = _run_sc_gather(b, lidar_phys, lin_flat).reshape(1, C, N)
        s3 = _tc1_call(b, g3, cam_bev_tokens, Wq, bq2, Wk)
        ot3, cs_b = _tc2_call(b, s3, cam_bev_tokens, gw3, Wv, Wo)
        delta_b, hits_b = _run_sc_scatter(b, ot3.reshape(C * N), lin_flat)
        d.append(delta_b.reshape(C, HG, WG, 8, 128))
        h.append(hits_b.reshape(HG, WG, 8, 128))
        cs.append(cs_b)

    return _combine_call(lidar_bev, d[0], d[1], range_alpha,
                         h[0], h[1], cs[0], cs[1])


# speedup vs baseline: 14.8518x; 1.1078x over previous
import functools
import math

import jax
import jax.numpy as jnp
from jax import lax
from jax.experimental import pallas as pl
from jax.experimental.pallas import tpu as pltpu
from jax.experimental.pallas import tpu_sc as plsc

B, C, H, W = 2, 128, 256, 256
HW = H * W
N, C_CAM = 20000, 128
HEADS, DHEAD = 4, 32
HD = HEADS * DHEAD
GAMMA = 0.08
SCALE = 1.0 / math.sqrt(DHEAD)

NTILES = 32
RPT = C // NTILES
NC = 2048
NSTEPS = (N + NC - 1) // NC
NP = NSTEPS * NC
HG = H // 8
WG = W // 128

_SC_MESH = plsc.VectorSubcoreMesh(core_axis_name="c", subcore_axis_name="s")
_SC_PARAMS = pltpu.CompilerParams(needs_layout_passes=False)


def _lin_body(ii_ref, jj_ref, lin_ref):
    i = jnp.clip(ii_ref[...], 0, H - 1)
    j = jnp.clip(jj_ref[...], 0, W - 1)
    lin_ref[...] = ((i // 8) * WG + j // 128) * 1024 \
        + (i % 8) * 128 + (j % 128)


def _lin_call(ii3, jj3):
    return pl.pallas_call(
        _lin_body,
        grid=(B,),
        in_specs=[
            pl.BlockSpec((1, 1, N), lambda b: (b, 0, 0)),
            pl.BlockSpec((1, 1, N), lambda b: (b, 0, 0)),
        ],
        out_specs=pl.BlockSpec((1, 1, N), lambda b: (b, 0, 0)),
        out_shape=jax.ShapeDtypeStruct((B, 1, N), jnp.int32),
    )(ii3, jj3)


def _sc_gather_body(b, lidar_hbm, lin_hbm, g_hbm, row_v, idx_v, out_v):
    cid = lax.axis_index("c")
    sid = lax.axis_index("s")
    wid = sid * 2 + cid
    pltpu.sync_copy(lin_hbm.at[pl.ds(b * N, N)], idx_v)

    for k in range(RPT):
        r = wid * RPT + k
        pltpu.sync_copy(lidar_hbm.at[pl.ds((b * C + r) * HW, HW)], row_v)

        @plsc.parallel_loop(0, N // 16, unroll=8)
        def _(i):
            off = i * 16
            ix = idx_v[pl.ds(off, 16)]
            out_v[pl.ds(off, 16)] = plsc.load_gather(row_v, [ix])

        pltpu.sync_copy(out_v, g_hbm.at[pl.ds(r * N, N)])


def _run_sc_gather(b, lidar_phys, lin_flat):
    fn = pl.kernel(
        functools.partial(_sc_gather_body, b),
        out_type=jax.ShapeDtypeStruct((C * N,), jnp.float32),
        mesh=_SC_MESH,
        compiler_params=_SC_PARAMS,
        scratch_types=[
            pltpu.VMEM((HW,), jnp.float32),
            pltpu.VMEM((N,), jnp.int32),
            pltpu.VMEM((N,), jnp.float32),
        ],
    )
    return fn(lidar_phys, lin_flat)


def _sc_scatter_body(b, ot_hbm, lin_hbm, delta_hbm, hits_hbm,
                     acc_v, idx_v, dat_v):
    cid = lax.axis_index("c")
    sid = lax.axis_index("s")
    wid = sid * 2 + cid
    pltpu.sync_copy(lin_hbm.at[pl.ds(b * N, N)], idx_v)
    zero16 = jnp.zeros((16,), jnp.float32)
    ones16 = jnp.ones((16,), jnp.float32)

    def scatter_add_loop():
        @plsc.parallel_loop(0, N // 16, unroll=8)
        def _(i):
            off = i * 16
            ix = idx_v[pl.ds(off, 16)]
            d = dat_v[pl.ds(off, 16)]
            plsc.addupdate_scatter(acc_v, [ix], d)

    def scatter_zero_loop():
        @plsc.parallel_loop(0, N // 16, unroll=8)
        def _(i):
            ix = idx_v[pl.ds(i * 16, 16)]
            plsc.store_scatter(acc_v, [ix], zero16)

    @pl.when(wid == 0)
    def _():
        @plsc.parallel_loop(0, HW // 16, unroll=8)
        def _(i):
            acc_v[pl.ds(i * 16, 16)] = zero16

        @plsc.parallel_loop(0, N // 16, unroll=8)
        def _(i):
            ix = idx_v[pl.ds(i * 16, 16)]
            plsc.addupdate_scatter(acc_v, [ix], ones16)

        pltpu.sync_copy(acc_v, hits_hbm)

    for k in range(RPT):
        r = wid * RPT + k
        pltpu.sync_copy(ot_hbm.at[pl.ds(r * N, N)], dat_v)
        scatter_zero_loop()
        scatter_add_loop()
        pltpu.sync_copy(acc_v, delta_hbm.at[pl.ds(r * HW, HW)])


def _run_sc_scatter(b, ot_flat, lin_flat):
    fn = pl.kernel(
        functools.partial(_sc_scatter_body, b),
        out_type=(
            jax.ShapeDtypeStruct((C * HW,), jnp.float32),
            jax.ShapeDtypeStruct((HW,), jnp.float32),
        ),
        mesh=_SC_MESH,
        compiler_params=_SC_PARAMS,
        scratch_types=[
            pltpu.VMEM((HW,), jnp.float32),
            pltpu.VMEM((N,), jnp.int32),
            pltpu.VMEM((N,), jnp.float32),
        ],
    )
    return fn(ot_flat, lin_flat)


def _head_onehot():
    col = lax.broadcasted_iota(jnp.int32, (HEADS, HD), 1) // DHEAD
    row = lax.broadcasted_iota(jnp.int32, (HEADS, HD), 0)
    return (col == row).astype(jnp.float32)


def _tc1_body(g_ref, tok_ref, wq_ref, bq_ref, wk_ref, s_ref):
    g = g_ref[0]
    tok = tok_ref[0]
    q = jnp.dot(wq_ref[...], g, preferred_element_type=jnp.float32) + bq_ref[...]
    k = lax.dot_general(wk_ref[...], tok, (((1,), (1,)), ((), ())),
                        preferred_element_type=jnp.float32)
    s = jnp.dot(_head_onehot(), q * k, preferred_element_type=jnp.float32)
    s_ref[0] = s * SCALE


def _tc1_call(b, g3, tok, Wq, bq2, Wk):
    return pl.pallas_call(
        _tc1_body,
        grid=(NSTEPS,),
        in_specs=[
            pl.BlockSpec((1, C, NC), lambda n: (0, 0, n)),
            pl.BlockSpec((1, NC, C_CAM), lambda n, _b=b: (_b, n, 0)),
            pl.BlockSpec((HD, C), lambda n: (0, 0)),
            pl.BlockSpec((HD, 1), lambda n: (0, 0)),
            pl.BlockSpec((HD, C_CAM), lambda n: (0, 0)),
        ],
        out_specs=pl.BlockSpec((1, HEADS, NC), lambda n: (0, 0, n)),
        out_shape=jax.ShapeDtypeStruct((1, HEADS, NP), jnp.float32),
    )(g3, tok, Wq, bq2, Wk)


def _tc2_body(s_ref, gw_ref, tok_ref, wv_ref, wo_ref, ot_ref, cs_ref, mz_ref):
    nstep = pl.program_id(0)
    lane_full = lax.broadcasted_iota(jnp.int32, (HEADS, NP), 1)

    @pl.when(nstep == 0)
    def _():
        s = s_ref[0]
        sm = jnp.where(lane_full < N, s, -jnp.inf)
        m = jnp.max(sm, axis=-1, keepdims=True)
        e = jnp.where(lane_full < N, jnp.exp(sm - m), 0.0)
        z = jnp.sum(e, axis=-1, keepdims=True)
        mz_ref[0:HEADS, 0:1] = m
        mz_ref[0:HEADS, 1:2] = z

    m = mz_ref[0:HEADS, 0:1]
    z = mz_ref[0:HEADS, 1:2]
    s_blk = s_ref[0, :, pl.ds(nstep * NC, NC)]
    lane = lax.broadcasted_iota(jnp.int32, (HEADS, NC), 1) + nstep * NC
    p = jnp.where(lane < N, jnp.exp(s_blk - m) / z, 0.0) * gw_ref[0]

    tok = tok_ref[0]
    v = lax.dot_general(wv_ref[...], tok, (((1,), (1,)), ((), ())),
                        preferred_element_type=jnp.float32)
    pe = lax.dot_general(_head_onehot(), p, (((0,), (0,)), ((), ())),
                         preferred_element_type=jnp.float32)
    lane2 = lax.broadcasted_iota(jnp.int32, (HD, NC), 1) + nstep * NC
    fused = jnp.where(lane2 < N, pe * v, 0.0)
    ot_ref[0] = jnp.dot(wo_ref[...], fused, preferred_element_type=jnp.float32)
    cs = jnp.dot(wo_ref[...], jnp.sum(fused, axis=1, keepdims=True),
                 preferred_element_type=jnp.float32)

    @pl.when(nstep == 0)
    def _():
        cs_ref[0] = cs

    @pl.when(nstep > 0)
    def _():
        cs_ref[0] += cs


def _tc2_call(b, s3, tok, gw3, Wv, Wo):
    return pl.pallas_call(
        _tc2_body,
        grid=(NSTEPS,),
        in_specs=[
            pl.BlockSpec((1, HEADS, NP), lambda n: (0, 0, 0)),
            pl.BlockSpec((1, 1, NC), lambda n, _b=b: (_b, 0, n)),
            pl.BlockSpec((1, NC, C_CAM), lambda n, _b=b: (_b, n, 0)),
            pl.BlockSpec((HD, C_CAM), lambda n: (0, 0)),
            pl.BlockSpec((C, HD), lambda n: (0, 0)),
        ],
        out_specs=[
            pl.BlockSpec((1, C, NC), lambda n: (0, 0, n)),
            pl.BlockSpec((1, C, 1), lambda n: (0, 0, 0)),
        ],
        out_shape=[
            jax.ShapeDtypeStruct((1, C, N), jnp.float32),
            jax.ShapeDtypeStruct((1, C, 1), jnp.float32),
        ],
        scratch_shapes=[pltpu.VMEM((8, 128), jnp.float32)],
    )(s3, gw3, tok, Wv, Wo)


def _tiles_to_pixels(x):
    return jnp.concatenate([x[..., g, :, :] for g in range(WG)], axis=-1)


def _combine_body(lid_ref, dl_ref, al_ref, ht_ref, hf_ref, cs_ref,
                  *rest):
    o_ref = rest[-1]
    hits_full = hf_ref[...]
    nhit = jnp.sum((hits_full > 0.0).astype(jnp.float32))
    mean = cs_ref[0].reshape(C, 1, 1) / (nhit + 1e-6)
    d = _tiles_to_pixels(dl_ref[:, 0])
    maskb = _tiles_to_pixels(ht_ref[0]) > 0.0
    dd = jnp.where(maskb[None], d - mean, 0.0)
    o_ref[0] = lid_ref[0] + dd * (al_ref[0] * GAMMA)


def _combine_call(b, lidar4, d_b, alpha4, h_b, cs_b, prev=None):
    in_specs = [
        pl.BlockSpec((1, C, 8, W), lambda p, _b=b: (_b, 0, p, 0)),
        pl.BlockSpec((C, 1, WG, 8, 128), lambda p: (0, p, 0, 0, 0)),
        pl.BlockSpec((1, 1, 8, W), lambda p: (0, 0, p, 0)),
        pl.BlockSpec((1, WG, 8, 128), lambda p: (p, 0, 0, 0)),
        pl.BlockSpec((HG, WG, 8, 128), lambda p: (0, 0, 0, 0)),
        pl.BlockSpec((1, C, 1), lambda p: (0, 0, 0)),
    ]
    args = [lidar4, d_b, alpha4, h_b, h_b, cs_b]
    aliases = {}
    if prev is not None:
        in_specs.append(pl.BlockSpec((1, C, 8, W), lambda p: (0, 0, 0, 0)))
        args.append(prev)
        aliases = {6: 0}
    return pl.pallas_call(
        _combine_body,
        grid=(HG,),
        in_specs=in_specs,
        out_specs=pl.BlockSpec((1, C, 8, W), lambda p, _b=b: (_b, 0, p, 0)),
        out_shape=jax.ShapeDtypeStruct((B, C, H, W), jnp.float32),
        input_output_aliases=aliases,
    )(*args)


def kernel(lidar_bev, cam_bev_tokens, cam_bev_indices, gate_weights,
           range_alpha, Wq, bq, Wk, Wv, Wo):
    lidar_phys = lidar_bev.reshape(B, C, HG, 8, WG, 128) \
        .transpose(0, 1, 2, 4, 3, 5).reshape(B * C * HW)
    ind = cam_bev_indices.astype(jnp.int32)
    ii3 = ind[..., 0].reshape(B, 1, N)
    jj3 = ind[..., 1].reshape(B, 1, N)
    gw3 = gate_weights.reshape(B, 1, N)
    bq2 = bq.reshape(HD, 1)

    lin_flat = _lin_call(ii3, jj3).reshape(B * N)

    out = None
    for b in range(B):
        g3 = _run_sc_gather(b, lidar_phys, lin_flat).reshape(1, C, N)
        s3 = _tc1_call(b, g3, cam_bev_tokens, Wq, bq2, Wk)
        ot3, cs_b = _tc2_call(b, s3, cam_bev_tokens, gw3, Wv, Wo)
        delta_b, hits_b = _run_sc_scatter(b, ot3.reshape(C * N), lin_flat)
        out = _combine_call(b, lidar_bev, delta_b.reshape(C, HG, WG, 8, 128),
                            range_alpha, hits_b.reshape(HG, WG, 8, 128),
                            cs_b, out)
    return out


# speedup vs baseline: 15.3625x; 1.0344x over previous
import functools
import math

import jax
import jax.numpy as jnp
from jax import lax
from jax.experimental import pallas as pl
from jax.experimental.pallas import tpu as pltpu
from jax.experimental.pallas import tpu_sc as plsc

B, C, H, W = 2, 128, 256, 256
HW = H * W
N, C_CAM = 20000, 128
HEADS, DHEAD = 4, 32
HD = HEADS * DHEAD
GAMMA = 0.08
SCALE = 1.0 / math.sqrt(DHEAD)

NTILES = 32
RPT = C // NTILES
NC = 2048
NSTEPS = (N + NC - 1) // NC
NP = NSTEPS * NC
NPAD = ((N + 255) // 256) * 256
HG = H // 8
WG = W // 128

_SC_MESH = plsc.VectorSubcoreMesh(core_axis_name="c", subcore_axis_name="s")
_SC_PARAMS = pltpu.CompilerParams(needs_layout_passes=False)


def _lin_body(ii_ref, jj_ref, lin_ref):
    i = jnp.clip(ii_ref[...], 0, H - 1)
    j = jnp.clip(jj_ref[...], 0, W - 1)
    lin_ref[...] = ((i // 8) * WG + j // 128) * 1024 \
        + (i % 8) * 128 + (j % 128)


def _lin_call(ii3, jj3):
    return pl.pallas_call(
        _lin_body,
        grid=(B,),
        in_specs=[
            pl.BlockSpec((1, 1, N), lambda b: (b, 0, 0)),
            pl.BlockSpec((1, 1, N), lambda b: (b, 0, 0)),
        ],
        out_specs=pl.BlockSpec((1, 1, N), lambda b: (b, 0, 0)),
        out_shape=jax.ShapeDtypeStruct((B, 1, N), jnp.int32),
    )(ii3, jj3)


def _sc_gather_body(b, lidar_hbm, lin_hbm, g_hbm, row_v, idx_v, out_v):
    cid = lax.axis_index("c")
    sid = lax.axis_index("s")
    wid = sid * 2 + cid
    pltpu.sync_copy(lin_hbm.at[pl.ds(b * N, N)], idx_v)

    for k in range(RPT):
        r = wid * RPT + k
        pltpu.sync_copy(lidar_hbm.at[pl.ds((b * C + r) * HW, HW)], row_v)

        @plsc.parallel_loop(0, N // 16, unroll=8)
        def _(i):
            off = i * 16
            ix = idx_v[pl.ds(off, 16)]
            out_v[pl.ds(off, 16)] = plsc.load_gather(row_v, [ix])

        pltpu.sync_copy(out_v, g_hbm.at[pl.ds(r * N, N)])


def _run_sc_gather(b, lidar_phys, lin_flat):
    fn = pl.kernel(
        functools.partial(_sc_gather_body, b),
        out_type=jax.ShapeDtypeStruct((C * N,), jnp.float32),
        mesh=_SC_MESH,
        compiler_params=_SC_PARAMS,
        scratch_types=[
            pltpu.VMEM((HW,), jnp.float32),
            pltpu.VMEM((N,), jnp.int32),
            pltpu.VMEM((N,), jnp.float32),
        ],
    )
    return fn(lidar_phys, lin_flat)


def _sc_scatter_body(b, ot_hbm, lin_hbm, delta_hbm, hits_hbm,
                     acc_v, idx_v, dat_v):
    cid = lax.axis_index("c")
    sid = lax.axis_index("s")
    wid = sid * 2 + cid
    pltpu.sync_copy(lin_hbm.at[pl.ds(b * N, N)], idx_v)
    zero16 = jnp.zeros((16,), jnp.float32)
    ones16 = jnp.ones((16,), jnp.float32)

    def scatter_add_loop():
        @plsc.parallel_loop(0, N // 32, unroll=4)
        def _(i):
            off = i * 32
            dd = dat_v[pl.ds(off, 32)]
            d0, d1 = plsc.unpack(dd, format=plsc.PackFormat.INTERLEAVED)
            ix0 = idx_v[pl.ds(off, 16)]
            ix1 = idx_v[pl.ds(off + 16, 16)]
            plsc.addupdate_scatter(acc_v, [ix0], d0)
            plsc.addupdate_scatter(acc_v, [ix1], d1)

    def scatter_zero_loop():
        @plsc.parallel_loop(0, N // 16, unroll=8)
        def _(i):
            ix = idx_v[pl.ds(i * 16, 16)]
            plsc.store_scatter(acc_v, [ix], zero16)

    @pl.when(wid == 0)
    def _():
        @plsc.parallel_loop(0, HW // 16, unroll=8)
        def _(i):
            acc_v[pl.ds(i * 16, 16)] = zero16

        @plsc.parallel_loop(0, N // 16, unroll=8)
        def _(i):
            ix = idx_v[pl.ds(i * 16, 16)]
            plsc.addupdate_scatter(acc_v, [ix], ones16)

        pltpu.sync_copy(acc_v, hits_hbm)

    for k in range(RPT):
        r = wid * RPT + k
        pltpu.sync_copy(ot_hbm.at[pl.ds(r * NPAD, NPAD)], dat_v)
        scatter_zero_loop()
        scatter_add_loop()
        pltpu.sync_copy(acc_v, delta_hbm.at[pl.ds(r * HW, HW)])


def _run_sc_scatter(b, ot_flat, lin_flat):
    fn = pl.kernel(
        functools.partial(_sc_scatter_body, b),
        out_type=(
            jax.ShapeDtypeStruct((C * HW,), jnp.float32),
            jax.ShapeDtypeStruct((HW,), jnp.float32),
        ),
        mesh=_SC_MESH,
        compiler_params=_SC_PARAMS,
        scratch_types=[
            pltpu.VMEM((HW,), jnp.float32),
            pltpu.VMEM((N,), jnp.int32),
            pltpu.VMEM((NPAD,), jnp.bfloat16),
        ],
    )
    return fn(ot_flat, lin_flat)


def _head_onehot():
    col = lax.broadcasted_iota(jnp.int32, (HEADS, HD), 1) // DHEAD
    row = lax.broadcasted_iota(jnp.int32, (HEADS, HD), 0)
    return (col == row).astype(jnp.float32)


def _tc1_body(g_ref, tok_ref, wq_ref, bq_ref, wk_ref, s_ref):
    g = g_ref[0]
    tok = tok_ref[0]
    q = jnp.dot(wq_ref[...], g, preferred_element_type=jnp.float32) + bq_ref[...]
    k = lax.dot_general(wk_ref[...], tok, (((1,), (1,)), ((), ())),
                        preferred_element_type=jnp.float32)
    s = jnp.dot(_head_onehot(), q * k, preferred_element_type=jnp.float32)
    s_ref[0] = s * SCALE


def _tc1_call(b, g3, tok, Wq, bq2, Wk):
    return pl.pallas_call(
        _tc1_body,
        grid=(NSTEPS,),
        in_specs=[
            pl.BlockSpec((1, C, NC), lambda n: (0, 0, n)),
            pl.BlockSpec((1, NC, C_CAM), lambda n, _b=b: (_b, n, 0)),
            pl.BlockSpec((HD, C), lambda n: (0, 0)),
            pl.BlockSpec((HD, 1), lambda n: (0, 0)),
            pl.BlockSpec((HD, C_CAM), lambda n: (0, 0)),
        ],
        out_specs=pl.BlockSpec((1, HEADS, NC), lambda n: (0, 0, n)),
        out_shape=jax.ShapeDtypeStruct((1, HEADS, NP), jnp.float32),
    )(g3, tok, Wq, bq2, Wk)


def _tc2_body(s_ref, gw_ref, tok_ref, wv_ref, wo_ref, ot_ref, cs_ref, mz_ref):
    nstep = pl.program_id(0)
    lane_full = lax.broadcasted_iota(jnp.int32, (HEADS, NP), 1)

    @pl.when(nstep == 0)
    def _():
        s = s_ref[0]
        sm = jnp.where(lane_full < N, s, -jnp.inf)
        m = jnp.max(sm, axis=-1, keepdims=True)
        e = jnp.where(lane_full < N, jnp.exp(sm - m), 0.0)
        z = jnp.sum(e, axis=-1, keepdims=True)
        mz_ref[0:HEADS, 0:1] = m
        mz_ref[0:HEADS, 1:2] = z

    m = mz_ref[0:HEADS, 0:1]
    z = mz_ref[0:HEADS, 1:2]
    s_blk = s_ref[0, :, pl.ds(nstep * NC, NC)]
    lane = lax.broadcasted_iota(jnp.int32, (HEADS, NC), 1) + nstep * NC
    p = jnp.where(lane < N, jnp.exp(s_blk - m) / z, 0.0) * gw_ref[0]

    tok = tok_ref[0]
    v = lax.dot_general(wv_ref[...], tok, (((1,), (1,)), ((), ())),
                        preferred_element_type=jnp.float32)
    pe = lax.dot_general(_head_onehot(), p, (((0,), (0,)), ((), ())),
                         preferred_element_type=jnp.float32)
    lane2 = lax.broadcasted_iota(jnp.int32, (HD, NC), 1) + nstep * NC
    fused = jnp.where(lane2 < N, pe * v, 0.0)
    ot_ref[0] = jnp.dot(wo_ref[...], fused,
                        preferred_element_type=jnp.float32).astype(jnp.bfloat16)
    cs = jnp.dot(wo_ref[...], jnp.sum(fused, axis=1, keepdims=True),
                 preferred_element_type=jnp.float32)

    @pl.when(nstep == 0)
    def _():
        cs_ref[0] = cs

    @pl.when(nstep > 0)
    def _():
        cs_ref[0] += cs


def _tc2_call(b, s3, tok, gw3, Wv, Wo):
    return pl.pallas_call(
        _tc2_body,
        grid=(NSTEPS,),
        in_specs=[
            pl.BlockSpec((1, HEADS, NP), lambda n: (0, 0, 0)),
            pl.BlockSpec((1, 1, NC), lambda n, _b=b: (_b, 0, n)),
            pl.BlockSpec((1, NC, C_CAM), lambda n, _b=b: (_b, n, 0)),
            pl.BlockSpec((HD, C_CAM), lambda n: (0, 0)),
            pl.BlockSpec((C, HD), lambda n: (0, 0)),
        ],
        out_specs=[
            pl.BlockSpec((1, C, NC), lambda n: (0, 0, n)),
            pl.BlockSpec((1, C, 1), lambda n: (0, 0, 0)),
        ],
        out_shape=[
            jax.ShapeDtypeStruct((1, C, NPAD), jnp.bfloat16),
            jax.ShapeDtypeStruct((1, C, 1), jnp.float32),
        ],
        scratch_shapes=[pltpu.VMEM((8, 128), jnp.float32)],
    )(s3, gw3, tok, Wv, Wo)


def _tiles_to_pixels(x):
    return jnp.concatenate([x[..., g, :, :] for g in range(WG)], axis=-1)


def _combine_body(lid_ref, dl_ref, al_ref, ht_ref, hf_ref, cs_ref,
                  *rest):
    o_ref = rest[-1]
    hits_full = hf_ref[...]
    nhit = jnp.sum((hits_full > 0.0).astype(jnp.float32))
    mean = cs_ref[0].reshape(C, 1, 1) / (nhit + 1e-6)
    d = _tiles_to_pixels(dl_ref[:, 0])
    maskb = _tiles_to_pixels(ht_ref[0]) > 0.0
    dd = jnp.where(maskb[None], d - mean, 0.0)
    o_ref[0] = lid_ref[0] + dd * (al_ref[0] * GAMMA)


def _combine_call(b, lidar4, d_b, alpha4, h_b, cs_b, prev=None):
    in_specs = [
        pl.BlockSpec((1, C, 8, W), lambda p, _b=b: (_b, 0, p, 0)),
        pl.BlockSpec((C, 1, WG, 8, 128), lambda p: (0, p, 0, 0, 0)),
        pl.BlockSpec((1, 1, 8, W), lambda p: (0, 0, p, 0)),
        pl.BlockSpec((1, WG, 8, 128), lambda p: (p, 0, 0, 0)),
        pl.BlockSpec((HG, WG, 8, 128), lambda p: (0, 0, 0, 0)),
        pl.BlockSpec((1, C, 1), lambda p: (0, 0, 0)),
    ]
    args = [lidar4, d_b, alpha4, h_b, h_b, cs_b]
    aliases = {}
    if prev is not None:
        in_specs.append(pl.BlockSpec((1, C, 8, W), lambda p: (0, 0, 0, 0)))
        args.append(prev)
        aliases = {6: 0}
    return pl.pallas_call(
        _combine_body,
        grid=(HG,),
        in_specs=in_specs,
        out_specs=pl.BlockSpec((1, C, 8, W), lambda p, _b=b: (_b, 0, p, 0)),
        out_shape=jax.ShapeDtypeStruct((B, C, H, W), jnp.float32),
        input_output_aliases=aliases,
    )(*args)


def kernel(lidar_bev, cam_bev_tokens, cam_bev_indices, gate_weights,
           range_alpha, Wq, bq, Wk, Wv, Wo):
    lidar_phys = lidar_bev.reshape(B, C, HG, 8, WG, 128) \
        .transpose(0, 1, 2, 4, 3, 5).reshape(B * C * HW)
    ind = cam_bev_indices.astype(jnp.int32)
    ii3 = ind[..., 0].reshape(B, 1, N)
    jj3 = ind[..., 1].reshape(B, 1, N)
    gw3 = gate_weights.reshape(B, 1, N)
    bq2 = bq.reshape(HD, 1)

    lin3 = _lin_call(ii3, jj3)
    lin_flat = lin3.reshape(B * N)
    lin_eo = lin3.reshape(B, N // 32, 16, 2).transpose(0, 1, 3, 2) \
        .reshape(B * N)

    out = None
    for b in range(B):
        g3 = _run_sc_gather(b, lidar_phys, lin_flat).reshape(1, C, N)
        s3 = _tc1_call(b, g3, cam_bev_tokens, Wq, bq2, Wk)
        ot3, cs_b = _tc2_call(b, s3, cam_bev_tokens, gw3, Wv, Wo)
        delta_b, hits_b = _run_sc_scatter(b, ot3.reshape(C * NPAD), lin_eo)
        out = _combine_call(b, lidar_bev, delta_b.reshape(C, HG, WG, 8, 128),
                            range_alpha, hits_b.reshape(HG, WG, 8, 128),
                            cs_b, out)
    return out
